# baseline (speedup 1.0000x reference)
import functools

import jax
import jax.numpy as jnp
from jax import lax
from jax.experimental import pallas as pl
from jax.experimental.pallas import tpu as pltpu
from jax.experimental.pallas import tpu_sc as plsc

BATCH = 16
SEQ_LEN = 1024
TOTAL = BATCH * SEQ_LEN
HALF = TOTAL // 2
EMB_DIM = 64
ENC_SIZE = 512
REL_DIM = 256
PAIR = 2 * EMB_DIM

NUM_WORKERS = 32
TPW = TOTAL // NUM_WORKERS
CHUNK = 128
ECH = CHUNK // 2
NCHUNK = TPW // CHUNK


def _emb_body(ipe, ipo, ice, ico, ise, iso,
              pos_t, cat_t, sense_t,
              out_pos, out_cat, out_sense,
              idx_v, pe_v, po_v, ce_v, co_v, se_v, so_v, gsem, wsem):
    wid = lax.axis_index("s") * 2 + lax.axis_index("c")
    pbase = wid * (TPW // 2)
    streams = ((ipe, ipo), (ice, ico), (ise, iso))
    staged = []
    for k, (ide, ido) in enumerate(streams):
        for j in range(NCHUNK):
            prows = pl.ds(pbase + j * ECH, ECH)
            staged.append(pltpu.async_copy(
                ide.at[prows], idx_v.at[(2 * k) * NCHUNK + j], gsem))
            staged.append(pltpu.async_copy(
                ido.at[prows], idx_v.at[(2 * k + 1) * NCHUNK + j], gsem))
    for h in staged:
        h.wait()
    bufs = ((pe_v, po_v), (ce_v, co_v), (se_v, so_v))
    tables = (pos_t, cat_t, sense_t)
    outs = (out_pos, out_cat, out_sense)
    for j in range(NCHUNK):
        prows = pl.ds(pbase + j * ECH, ECH)
        gathers = []
        for k in range(3):
            gathers.append(pltpu.async_copy(
                tables[k].at[idx_v.at[(2 * k) * NCHUNK + j]], bufs[k][0], gsem))
            gathers.append(pltpu.async_copy(
                tables[k].at[idx_v.at[(2 * k + 1) * NCHUNK + j]], bufs[k][1],
                gsem))
        for h in gathers:
            h.wait()
        writes = []
        for k in range(3):
            writes.append(pltpu.async_copy(
                bufs[k][0], outs[k].at[prows, pl.ds(0, EMB_DIM)], wsem))
            writes.append(pltpu.async_copy(
                bufs[k][1], outs[k].at[prows, pl.ds(EMB_DIM, EMB_DIM)], wsem))
        for h in writes:
            h.wait()


_emb_gather = functools.partial(
    pl.kernel,
    mesh=plsc.VectorSubcoreMesh(core_axis_name="c", subcore_axis_name="s"),
    out_type=(
        jax.ShapeDtypeStruct((HALF, PAIR), jnp.float32),
        jax.ShapeDtypeStruct((HALF, PAIR), jnp.float32),
        jax.ShapeDtypeStruct((HALF, PAIR), jnp.float32),
    ),
    scratch_types=[
        pltpu.VMEM((24, ECH), jnp.int32),
        pltpu.VMEM((ECH, EMB_DIM), jnp.float32),
        pltpu.VMEM((ECH, EMB_DIM), jnp.float32),
        pltpu.VMEM((ECH, EMB_DIM), jnp.float32),
        pltpu.VMEM((ECH, EMB_DIM), jnp.float32),
        pltpu.VMEM((ECH, EMB_DIM), jnp.float32),
        pltpu.VMEM((ECH, EMB_DIM), jnp.float32),
        pltpu.SemaphoreType.DMA,
        pltpu.SemaphoreType.DMA,
    ],
    compiler_params=pltpu.CompilerParams(use_tc_tiling_on_sc=False),
)(_emb_body)


def _head_gather_body(ihe, iho, head_t, out_head,
                      idx_v, he_v, ho_v, gsem, wsem):
    wid = lax.axis_index("s") * 2 + lax.axis_index("c")
    pbase = wid * (TPW // 2)
    staged = []
    for j in range(NCHUNK):
        prows = pl.ds(pbase + j * ECH, ECH)
        staged.append(pltpu.async_copy(ihe.at[prows], idx_v.at[2 * j], gsem))
        staged.append(pltpu.async_copy(iho.at[prows], idx_v.at[2 * j + 1],
                                       gsem))
    for h in staged:
        h.wait()
    for j in range(NCHUNK):
        prows = pl.ds(pbase + j * ECH, ECH)
        g = (pltpu.async_copy(head_t.at[idx_v.at[2 * j]], he_v, gsem),
             pltpu.async_copy(head_t.at[idx_v.at[2 * j + 1]], ho_v, gsem))
        for h in g:
            h.wait()
        w = (pltpu.async_copy(he_v, out_head.at[prows, pl.ds(0, REL_DIM)],
                              wsem),
             pltpu.async_copy(ho_v, out_head.at[prows, pl.ds(REL_DIM,
                                                             REL_DIM)], wsem))
        for h in w:
            h.wait()


_head_gather = functools.partial(
    pl.kernel,
    mesh=plsc.VectorSubcoreMesh(core_axis_name="c", subcore_axis_name="s"),
    out_type=jax.ShapeDtypeStruct((HALF, 2 * REL_DIM), jnp.float32),
    scratch_types=[
        pltpu.VMEM((8, ECH), jnp.int32),
        pltpu.VMEM((ECH, REL_DIM), jnp.float32),
        pltpu.VMEM((ECH, REL_DIM), jnp.float32),
        pltpu.SemaphoreType.DMA,
        pltpu.SemaphoreType.DMA,
    ],
)(_head_gather_body)


BM = 1024


def _head_body(x_ref, w_ref, o_ref):
    o_ref[...] = jnp.dot(x_ref[...], w_ref[...],
                         preferred_element_type=jnp.float32)


def _head_proj(x, wh):
    return pl.pallas_call(
        _head_body,
        grid=(TOTAL // BM,),
        in_specs=[
            pl.BlockSpec((BM, ENC_SIZE), lambda i: (i, 0)),
            pl.BlockSpec((ENC_SIZE, REL_DIM), lambda i: (0, 0)),
        ],
        out_specs=pl.BlockSpec((BM, REL_DIM), lambda i: (i, 0)),
        out_shape=jax.ShapeDtypeStruct((TOTAL, REL_DIM), jnp.float32),
    )(x, wh)


BM2 = BM // 2


def _mm_body(gp_hbm, gc_hbm, gs_hbm, gh_hbm, wp_ref, wc_ref, ws_ref,
             b_ref, o_ref, gp_v, gc_v, gs_v, gh_v, sem):
    i = pl.program_id(0)
    rows = pl.ds(i * BM2, BM2)
    cps = (pltpu.async_copy(gp_hbm.at[rows, :], gp_v, sem),
           pltpu.async_copy(gc_hbm.at[rows, :], gc_v, sem),
           pltpu.async_copy(gs_hbm.at[rows, :], gs_v, sem),
           pltpu.async_copy(gh_hbm.at[rows, :], gh_v, sem))
    for h in cps:
        h.wait()
    acc = gh_v[...] + b_ref[...]
    acc += jnp.dot(gp_v[...], wp_ref[...],
                   preferred_element_type=jnp.float32)
    acc += jnp.dot(gc_v[...], wc_ref[...],
                   preferred_element_type=jnp.float32)
    acc += jnp.dot(gs_v[...], ws_ref[...],
                   preferred_element_type=jnp.float32)
    o_ref[...] = jnp.maximum(acc, 0.0).reshape(BM, REL_DIM)


def _matmul(gp, gc, gs, gh, wp2, wc2, ws2, b2):
    any_spec = pl.BlockSpec(memory_space=pl.ANY)
    w_spec = pl.BlockSpec((PAIR, 2 * REL_DIM), lambda i: (0, 0))
    return pl.pallas_call(
        _mm_body,
        grid=(HALF // BM2,),
        in_specs=[
            any_spec, any_spec, any_spec, any_spec,
            w_spec, w_spec, w_spec,
            pl.BlockSpec((1, 2 * REL_DIM), lambda i: (0, 0)),
        ],
        out_specs=pl.BlockSpec((BM, REL_DIM), lambda i: (i, 0)),
        out_shape=jax.ShapeDtypeStruct((TOTAL, REL_DIM), jnp.float32),
        scratch_shapes=[
            pltpu.VMEM((BM2, PAIR), jnp.float32),
            pltpu.VMEM((BM2, PAIR), jnp.float32),
            pltpu.VMEM((BM2, PAIR), jnp.float32),
            pltpu.VMEM((BM2, 2 * REL_DIM), jnp.float32),
            pltpu.SemaphoreType.DMA,
        ],
    )(gp, gc, gs, gh, wp2, wc2, ws2, b2)


def _blockdiag(w):
    z = jnp.zeros((EMB_DIM, REL_DIM), jnp.float32)
    return jnp.concatenate(
        [jnp.concatenate([w, z], axis=1),
         jnp.concatenate([z, w], axis=1)], axis=0)


def kernel(input_data, index, src_enc_data, pos_table, cat_table, sense_table,
           W, b, lengths):
    ids = input_data.astype(jnp.int32)
    t = jnp.arange(TOTAL, dtype=jnp.int32)
    flat_idx = (t // SEQ_LEN) * SEQ_LEN + index.astype(jnp.int32)
    ipe, ice, ise = ids[0::2, 0], ids[0::2, 1], ids[0::2, 2]
    ipo, ico, iso = ids[1::2, 0], ids[1::2, 1], ids[1::2, 2]
    ihe, iho = flat_idx[0::2], flat_idx[1::2]
    wp = W[:EMB_DIM]
    wc = W[EMB_DIM:2 * EMB_DIM]
    ws = W[2 * EMB_DIM:3 * EMB_DIM]
    wh = W[3 * EMB_DIM:]
    hproj = _head_proj(src_enc_data, wh)
    gp, gc, gs = _emb_gather(ipe, ipo, ice, ico, ise, iso,
                             pos_table, cat_table, sense_table)
    gh = _head_gather(ihe, iho, hproj)
    return _matmul(gp, gc, gs, gh, _blockdiag(wp), _blockdiag(wc),
                   _blockdiag(ws),
                   jnp.concatenate([b, b]).reshape(1, 2 * REL_DIM))


# speedup vs baseline: 1.1000x; 1.1000x over previous
import functools

import jax
import jax.numpy as jnp
from jax import lax
from jax.experimental import pallas as pl
from jax.experimental.pallas import tpu as pltpu
from jax.experimental.pallas import tpu_sc as plsc

BATCH = 16
SEQ_LEN = 1024
TOTAL = BATCH * SEQ_LEN
HALF = TOTAL // 2
EMB_DIM = 64
ENC_SIZE = 512
REL_DIM = 256
PAIR = 2 * EMB_DIM

NUM_WORKERS = 32
TPW = TOTAL // NUM_WORKERS
CHUNK = 128
ECH = CHUNK // 2
NCHUNK = TPW // CHUNK


def _emb_body(ipe, ipo, ice, ico, ise, iso,
              pos_t, cat_t, sense_t,
              out_pos, out_cat, out_sense,
              idx_v, pe_v, po_v, ce_v, co_v, se_v, so_v, gsem, wsem):
    wid = lax.axis_index("s") * 2 + lax.axis_index("c")
    pbase = wid * (TPW // 2)
    streams = ((ipe, ipo), (ice, ico), (ise, iso))
    staged = []
    for k, (ide, ido) in enumerate(streams):
        for j in range(NCHUNK):
            prows = pl.ds(pbase + j * ECH, ECH)
            staged.append(pltpu.async_copy(
                ide.at[prows], idx_v.at[(2 * k) * NCHUNK + j], gsem))
            staged.append(pltpu.async_copy(
                ido.at[prows], idx_v.at[(2 * k + 1) * NCHUNK + j], gsem))
    for h in staged:
        h.wait()
    bufs = ((pe_v, po_v), (ce_v, co_v), (se_v, so_v))
    tables = (pos_t, cat_t, sense_t)
    outs = (out_pos, out_cat, out_sense)
    for j in range(NCHUNK):
        prows = pl.ds(pbase + j * ECH, ECH)
        gathers = []
        for k in range(3):
            gathers.append(pltpu.async_copy(
                tables[k].at[idx_v.at[(2 * k) * NCHUNK + j]], bufs[k][0], gsem))
            gathers.append(pltpu.async_copy(
                tables[k].at[idx_v.at[(2 * k + 1) * NCHUNK + j]], bufs[k][1],
                gsem))
        for h in gathers:
            h.wait()
        writes = []
        for k in range(3):
            writes.append(pltpu.async_copy(
                bufs[k][0], outs[k].at[prows, pl.ds(0, EMB_DIM)], wsem))
            writes.append(pltpu.async_copy(
                bufs[k][1], outs[k].at[prows, pl.ds(EMB_DIM, EMB_DIM)], wsem))
        for h in writes:
            h.wait()


_emb_gather = functools.partial(
    pl.kernel,
    mesh=plsc.VectorSubcoreMesh(core_axis_name="c", subcore_axis_name="s"),
    out_type=(
        jax.ShapeDtypeStruct((HALF, PAIR), jnp.float32),
        jax.ShapeDtypeStruct((HALF, PAIR), jnp.float32),
        jax.ShapeDtypeStruct((HALF, PAIR), jnp.float32),
    ),
    scratch_types=[
        pltpu.VMEM((24, ECH), jnp.int32),
        pltpu.VMEM((ECH, EMB_DIM), jnp.float32),
        pltpu.VMEM((ECH, EMB_DIM), jnp.float32),
        pltpu.VMEM((ECH, EMB_DIM), jnp.float32),
        pltpu.VMEM((ECH, EMB_DIM), jnp.float32),
        pltpu.VMEM((ECH, EMB_DIM), jnp.float32),
        pltpu.VMEM((ECH, EMB_DIM), jnp.float32),
        pltpu.SemaphoreType.DMA,
        pltpu.SemaphoreType.DMA,
    ],
    compiler_params=pltpu.CompilerParams(use_tc_tiling_on_sc=False),
)(_emb_body)


def _head_gather_body(ihe, iho, head_t, out_head,
                      idx_v, he_v, ho_v, gsem, wsem):
    wid = lax.axis_index("s") * 2 + lax.axis_index("c")
    pbase = wid * (TPW // 2)
    staged = []
    for j in range(NCHUNK):
        prows = pl.ds(pbase + j * ECH, ECH)
        staged.append(pltpu.async_copy(ihe.at[prows], idx_v.at[2 * j], gsem))
        staged.append(pltpu.async_copy(iho.at[prows], idx_v.at[2 * j + 1],
                                       gsem))
    for h in staged:
        h.wait()
    for j in range(NCHUNK):
        prows = pl.ds(pbase + j * ECH, ECH)
        g = (pltpu.async_copy(head_t.at[idx_v.at[2 * j]], he_v, gsem),
             pltpu.async_copy(head_t.at[idx_v.at[2 * j + 1]], ho_v, gsem))
        for h in g:
            h.wait()
        w = (pltpu.async_copy(he_v, out_head.at[prows, pl.ds(0, REL_DIM)],
                              wsem),
             pltpu.async_copy(ho_v, out_head.at[prows, pl.ds(REL_DIM,
                                                             REL_DIM)], wsem))
        for h in w:
            h.wait()


_head_gather = functools.partial(
    pl.kernel,
    mesh=plsc.VectorSubcoreMesh(core_axis_name="c", subcore_axis_name="s"),
    out_type=jax.ShapeDtypeStruct((HALF, 2 * REL_DIM), jnp.float32),
    scratch_types=[
        pltpu.VMEM((8, ECH), jnp.int32),
        pltpu.VMEM((ECH, REL_DIM), jnp.float32),
        pltpu.VMEM((ECH, REL_DIM), jnp.float32),
        pltpu.SemaphoreType.DMA,
        pltpu.SemaphoreType.DMA,
    ],
)(_head_gather_body)


BM = 1024


def _head_body(x_ref, w_ref, o_ref):
    o_ref[...] = jnp.dot(x_ref[...], w_ref[...],
                         preferred_element_type=jnp.float32)


def _head_proj(x, wh):
    return pl.pallas_call(
        _head_body,
        grid=(TOTAL // BM,),
        in_specs=[
            pl.BlockSpec((BM, ENC_SIZE), lambda i: (i, 0)),
            pl.BlockSpec((ENC_SIZE, REL_DIM), lambda i: (0, 0)),
        ],
        out_specs=pl.BlockSpec((BM, REL_DIM), lambda i: (i, 0)),
        out_shape=jax.ShapeDtypeStruct((TOTAL, REL_DIM), jnp.float32),
    )(x, wh)


BM2 = BM // 2


def _mm_body(gp_ref, gc_ref, gs_ref, gh_ref, wp_ref, wc_ref, ws_ref,
             b_ref, o_ref):
    acc = gh_ref[...] + b_ref[...]
    acc += jnp.dot(gp_ref[...], wp_ref[...],
                   preferred_element_type=jnp.float32)
    acc += jnp.dot(gc_ref[...], wc_ref[...],
                   preferred_element_type=jnp.float32)
    acc += jnp.dot(gs_ref[...], ws_ref[...],
                   preferred_element_type=jnp.float32)
    o_ref[...] = jnp.maximum(acc, 0.0).reshape(BM, REL_DIM)


def _matmul(gp, gc, gs, gh, wp2, wc2, ws2, b2):
    pair_spec = pl.BlockSpec((BM2, PAIR), lambda i: (i, 0))
    w_spec = pl.BlockSpec((PAIR, 2 * REL_DIM), lambda i: (0, 0))
    return pl.pallas_call(
        _mm_body,
        grid=(HALF // BM2,),
        in_specs=[
            pair_spec, pair_spec, pair_spec,
            pl.BlockSpec((BM2, 2 * REL_DIM), lambda i: (i, 0)),
            w_spec, w_spec, w_spec,
            pl.BlockSpec((1, 2 * REL_DIM), lambda i: (0, 0)),
        ],
        out_specs=pl.BlockSpec((BM, REL_DIM), lambda i: (i, 0)),
        out_shape=jax.ShapeDtypeStruct((TOTAL, REL_DIM), jnp.float32),
    )(gp, gc, gs, gh, wp2, wc2, ws2, b2)


def _blockdiag(w):
    z = jnp.zeros((EMB_DIM, REL_DIM), jnp.float32)
    return jnp.concatenate(
        [jnp.concatenate([w, z], axis=1),
         jnp.concatenate([z, w], axis=1)], axis=0)


def kernel(input_data, index, src_enc_data, pos_table, cat_table, sense_table,
           W, b, lengths):
    ids = input_data.astype(jnp.int32)
    t = jnp.arange(TOTAL, dtype=jnp.int32)
    flat_idx = (t // SEQ_LEN) * SEQ_LEN + index.astype(jnp.int32)
    ipe, ice, ise = ids[0::2, 0], ids[0::2, 1], ids[0::2, 2]
    ipo, ico, iso = ids[1::2, 0], ids[1::2, 1], ids[1::2, 2]
    ihe, iho = flat_idx[0::2], flat_idx[1::2]
    wp = W[:EMB_DIM]
    wc = W[EMB_DIM:2 * EMB_DIM]
    ws = W[2 * EMB_DIM:3 * EMB_DIM]
    wh = W[3 * EMB_DIM:]
    hproj = _head_proj(src_enc_data, wh)
    gp, gc, gs = _emb_gather(ipe, ipo, ice, ico, ise, iso,
                             pos_table, cat_table, sense_table)
    gh = _head_gather(ihe, iho, hproj)
    return _matmul(gp, gc, gs, gh, _blockdiag(wp), _blockdiag(wc),
                   _blockdiag(ws),
                   jnp.concatenate([b, b]).reshape(1, 2 * REL_DIM))


# speedup vs baseline: 1.1014x; 1.0013x over previous
import functools

import jax
import jax.numpy as jnp
from jax import lax
from jax.experimental import pallas as pl
from jax.experimental.pallas import tpu as pltpu
from jax.experimental.pallas import tpu_sc as plsc

BATCH = 16
SEQ_LEN = 1024
TOTAL = BATCH * SEQ_LEN
HALF = TOTAL // 2
EMB_DIM = 64
ENC_SIZE = 512
REL_DIM = 256
PAIR = 2 * EMB_DIM

NUM_WORKERS = 32
TPW = TOTAL // NUM_WORKERS
CHUNK = 128
ECH = CHUNK // 2
NCHUNK = TPW // CHUNK


def _emb_body(ipe, ipo, ice, ico, ise, iso,
              pos_t, cat_t, sense_t,
              out_pos, out_cat, out_sense,
              idx_v, pe_v, po_v, ce_v, co_v, se_v, so_v, gsem, wsem):
    wid = lax.axis_index("s") * 2 + lax.axis_index("c")
    pbase = wid * (TPW // 2)
    streams = ((ipe, ipo), (ice, ico), (ise, iso))
    staged = []
    for k, (ide, ido) in enumerate(streams):
        for j in range(NCHUNK):
            prows = pl.ds(pbase + j * ECH, ECH)
            staged.append(pltpu.async_copy(
                ide.at[prows], idx_v.at[(2 * k) * NCHUNK + j], gsem))
            staged.append(pltpu.async_copy(
                ido.at[prows], idx_v.at[(2 * k + 1) * NCHUNK + j], gsem))
    for h in staged:
        h.wait()
    bufs = ((pe_v, po_v), (ce_v, co_v), (se_v, so_v))
    tables = (pos_t, cat_t, sense_t)
    outs = (out_pos, out_cat, out_sense)
    for j in range(NCHUNK):
        prows = pl.ds(pbase + j * ECH, ECH)
        gathers = []
        for k in range(3):
            gathers.append(pltpu.async_copy(
                tables[k].at[idx_v.at[(2 * k) * NCHUNK + j]], bufs[k][0], gsem))
            gathers.append(pltpu.async_copy(
                tables[k].at[idx_v.at[(2 * k + 1) * NCHUNK + j]], bufs[k][1],
                gsem))
        for h in gathers:
            h.wait()
        writes = []
        for k in range(3):
            writes.append(pltpu.async_copy(
                bufs[k][0], outs[k].at[prows, pl.ds(0, EMB_DIM)], wsem))
            writes.append(pltpu.async_copy(
                bufs[k][1], outs[k].at[prows, pl.ds(EMB_DIM, EMB_DIM)], wsem))
        for h in writes:
            h.wait()


_emb_gather = functools.partial(
    pl.kernel,
    mesh=plsc.VectorSubcoreMesh(core_axis_name="c", subcore_axis_name="s"),
    out_type=(
        jax.ShapeDtypeStruct((HALF, PAIR), jnp.float32),
        jax.ShapeDtypeStruct((HALF, PAIR), jnp.float32),
        jax.ShapeDtypeStruct((HALF, PAIR), jnp.float32),
    ),
    scratch_types=[
        pltpu.VMEM((24, ECH), jnp.int32),
        pltpu.VMEM((ECH, EMB_DIM), jnp.float32),
        pltpu.VMEM((ECH, EMB_DIM), jnp.float32),
        pltpu.VMEM((ECH, EMB_DIM), jnp.float32),
        pltpu.VMEM((ECH, EMB_DIM), jnp.float32),
        pltpu.VMEM((ECH, EMB_DIM), jnp.float32),
        pltpu.VMEM((ECH, EMB_DIM), jnp.float32),
        pltpu.SemaphoreType.DMA,
        pltpu.SemaphoreType.DMA,
    ],
    compiler_params=pltpu.CompilerParams(use_tc_tiling_on_sc=False),
)(_emb_body)


def _head_gather_body(ihe, iho, hpa, hpb, oea, oeb, ooa, oob,
                      idx_v, ea_v, eb_v, oa_v, ob_v, gsem, wsem):
    wid = lax.axis_index("s") * 2 + lax.axis_index("c")
    pbase = wid * (TPW // 2)
    staged = []
    for j in range(NCHUNK):
        prows = pl.ds(pbase + j * ECH, ECH)
        staged.append(pltpu.async_copy(ihe.at[prows], idx_v.at[2 * j], gsem))
        staged.append(pltpu.async_copy(iho.at[prows], idx_v.at[2 * j + 1],
                                       gsem))
    for h in staged:
        h.wait()
    for j in range(NCHUNK):
        prows = pl.ds(pbase + j * ECH, ECH)
        g = (pltpu.async_copy(hpa.at[idx_v.at[2 * j]], ea_v, gsem),
             pltpu.async_copy(hpb.at[idx_v.at[2 * j]], eb_v, gsem),
             pltpu.async_copy(hpa.at[idx_v.at[2 * j + 1]], oa_v, gsem),
             pltpu.async_copy(hpb.at[idx_v.at[2 * j + 1]], ob_v, gsem))
        for h in g:
            h.wait()
        w = (pltpu.async_copy(ea_v, oea.at[prows], wsem),
             pltpu.async_copy(eb_v, oeb.at[prows], wsem),
             pltpu.async_copy(oa_v, ooa.at[prows], wsem),
             pltpu.async_copy(ob_v, oob.at[prows], wsem))
        for h in w:
            h.wait()


_head_gather = functools.partial(
    pl.kernel,
    mesh=plsc.VectorSubcoreMesh(core_axis_name="c", subcore_axis_name="s"),
    out_type=(
        jax.ShapeDtypeStruct((HALF, PAIR), jnp.float32),
        jax.ShapeDtypeStruct((HALF, PAIR), jnp.float32),
        jax.ShapeDtypeStruct((HALF, PAIR), jnp.float32),
        jax.ShapeDtypeStruct((HALF, PAIR), jnp.float32),
    ),
    scratch_types=[
        pltpu.VMEM((8, ECH), jnp.int32),
        pltpu.VMEM((ECH, PAIR), jnp.float32),
        pltpu.VMEM((ECH, PAIR), jnp.float32),
        pltpu.VMEM((ECH, PAIR), jnp.float32),
        pltpu.VMEM((ECH, PAIR), jnp.float32),
        pltpu.SemaphoreType.DMA,
        pltpu.SemaphoreType.DMA,
    ],
)(_head_gather_body)


BM = 1024


def _head_body(x_ref, w_ref, oa_ref, ob_ref):
    acc = jnp.dot(x_ref[...], w_ref[...], preferred_element_type=jnp.float32)
    oa_ref[...] = acc[:, :PAIR]
    ob_ref[...] = acc[:, PAIR:]


def _head_proj(x, wh):
    return pl.pallas_call(
        _head_body,
        grid=(TOTAL // BM,),
        in_specs=[
            pl.BlockSpec((BM, ENC_SIZE), lambda i: (i, 0)),
            pl.BlockSpec((ENC_SIZE, REL_DIM), lambda i: (0, 0)),
        ],
        out_specs=[
            pl.BlockSpec((BM, PAIR), lambda i: (i, 0)),
            pl.BlockSpec((BM, PAIR), lambda i: (i, 0)),
        ],
        out_shape=[
            jax.ShapeDtypeStruct((TOTAL, PAIR), jnp.float32),
            jax.ShapeDtypeStruct((TOTAL, PAIR), jnp.float32),
        ],
    )(x, wh)


BM2 = BM // 2


def _mm_body(gp_ref, gc_ref, gs_ref, ea_ref, eb_ref, oa_ref, ob_ref,
             wp_ref, wc_ref, ws_ref, b_ref, o_ref):
    acc = jnp.concatenate(
        [ea_ref[...], eb_ref[...], oa_ref[...], ob_ref[...]], axis=1)
    acc += b_ref[...]
    acc += jnp.dot(gp_ref[...], wp_ref[...],
                   preferred_element_type=jnp.float32)
    acc += jnp.dot(gc_ref[...], wc_ref[...],
                   preferred_element_type=jnp.float32)
    acc += jnp.dot(gs_ref[...], ws_ref[...],
                   preferred_element_type=jnp.float32)
    o_ref[...] = jnp.maximum(acc, 0.0).reshape(BM, REL_DIM)


def _matmul(gp, gc, gs, ghs, wp2, wc2, ws2, b2):
    pair_spec = pl.BlockSpec((BM2, PAIR), lambda i: (i, 0))
    w_spec = pl.BlockSpec((PAIR, 2 * REL_DIM), lambda i: (0, 0))
    return pl.pallas_call(
        _mm_body,
        grid=(HALF // BM2,),
        in_specs=[
            pair_spec, pair_spec, pair_spec,
            pair_spec, pair_spec, pair_spec, pair_spec,
            w_spec, w_spec, w_spec,
            pl.BlockSpec((1, 2 * REL_DIM), lambda i: (0, 0)),
        ],
        out_specs=pl.BlockSpec((BM, REL_DIM), lambda i: (i, 0)),
        out_shape=jax.ShapeDtypeStruct((TOTAL, REL_DIM), jnp.float32),
    )(gp, gc, gs, *ghs, wp2, wc2, ws2, b2)


def _blockdiag(w):
    z = jnp.zeros((EMB_DIM, REL_DIM), jnp.float32)
    return jnp.concatenate(
        [jnp.concatenate([w, z], axis=1),
         jnp.concatenate([z, w], axis=1)], axis=0)


def kernel(input_data, index, src_enc_data, pos_table, cat_table, sense_table,
           W, b, lengths):
    ids = input_data.astype(jnp.int32)
    t = jnp.arange(TOTAL, dtype=jnp.int32)
    flat_idx = (t // SEQ_LEN) * SEQ_LEN + index.astype(jnp.int32)
    ipe, ice, ise = ids[0::2, 0], ids[0::2, 1], ids[0::2, 2]
    ipo, ico, iso = ids[1::2, 0], ids[1::2, 1], ids[1::2, 2]
    ihe, iho = flat_idx[0::2], flat_idx[1::2]
    wp = W[:EMB_DIM]
    wc = W[EMB_DIM:2 * EMB_DIM]
    ws = W[2 * EMB_DIM:3 * EMB_DIM]
    wh = W[3 * EMB_DIM:]
    hpa, hpb = _head_proj(src_enc_data, wh)
    gp, gc, gs = _emb_gather(ipe, ipo, ice, ico, ise, iso,
                             pos_table, cat_table, sense_table)
    ghs = _head_gather(ihe, iho, hpa, hpb)
    return _matmul(gp, gc, gs, ghs, _blockdiag(wp), _blockdiag(wc),
                   _blockdiag(ws),
                   jnp.concatenate([b, b]).reshape(1, 2 * REL_DIM))


# speedup vs baseline: 1.1571x; 1.0506x over previous
import functools

import jax
import jax.numpy as jnp
from jax import lax
from jax.experimental import pallas as pl
from jax.experimental.pallas import tpu as pltpu
from jax.experimental.pallas import tpu_sc as plsc

BATCH = 16
SEQ_LEN = 1024
TOTAL = BATCH * SEQ_LEN
HALF = TOTAL // 2
EMB_DIM = 64
ENC_SIZE = 512
REL_DIM = 256
PAIR = 2 * EMB_DIM

NUM_WORKERS = 32
TPW = TOTAL // NUM_WORKERS
CHUNK = 128
ECH = CHUNK // 2
NCHUNK = TPW // CHUNK


def _emb_body(ipe, ipo, ice, ico, ise, iso,
              pos_t, cat_t, sense_t,
              out_pos, out_cat, out_sense,
              idx_v, pe_v, po_v, ce_v, co_v, se_v, so_v, gsem, wsem):
    wid = lax.axis_index("s") * 2 + lax.axis_index("c")
    pbase = wid * (TPW // 2)
    streams = ((ipe, ipo), (ice, ico), (ise, iso))
    staged = []
    for k, (ide, ido) in enumerate(streams):
        for j in range(NCHUNK):
            prows = pl.ds(pbase + j * ECH, ECH)
            staged.append(pltpu.async_copy(
                ide.at[prows], idx_v.at[(2 * k) * NCHUNK + j], gsem))
            staged.append(pltpu.async_copy(
                ido.at[prows], idx_v.at[(2 * k + 1) * NCHUNK + j], gsem))
    for h in staged:
        h.wait()
    bufs = ((pe_v, po_v), (ce_v, co_v), (se_v, so_v))
    tables = (pos_t, cat_t, sense_t)
    outs = (out_pos, out_cat, out_sense)
    for j in range(NCHUNK):
        prows = pl.ds(pbase + j * ECH, ECH)
        gathers = []
        for k in range(3):
            gathers.append(pltpu.async_copy(
                tables[k].at[idx_v.at[(2 * k) * NCHUNK + j]], bufs[k][0], gsem))
            gathers.append(pltpu.async_copy(
                tables[k].at[idx_v.at[(2 * k + 1) * NCHUNK + j]], bufs[k][1],
                gsem))
        for h in gathers:
            h.wait()
        writes = []
        for k in range(3):
            writes.append(pltpu.async_copy(
                bufs[k][0], outs[k].at[prows, pl.ds(0, EMB_DIM)], wsem))
            writes.append(pltpu.async_copy(
                bufs[k][1], outs[k].at[prows, pl.ds(EMB_DIM, EMB_DIM)], wsem))
        for h in writes:
            h.wait()


_emb_gather = functools.partial(
    pl.kernel,
    mesh=plsc.VectorSubcoreMesh(core_axis_name="c", subcore_axis_name="s"),
    out_type=(
        jax.ShapeDtypeStruct((HALF, PAIR), jnp.float32),
        jax.ShapeDtypeStruct((HALF, PAIR), jnp.float32),
        jax.ShapeDtypeStruct((HALF, PAIR), jnp.float32),
    ),
    scratch_types=[
        pltpu.VMEM((24, ECH), jnp.int32),
        pltpu.VMEM((ECH, EMB_DIM), jnp.float32),
        pltpu.VMEM((ECH, EMB_DIM), jnp.float32),
        pltpu.VMEM((ECH, EMB_DIM), jnp.float32),
        pltpu.VMEM((ECH, EMB_DIM), jnp.float32),
        pltpu.VMEM((ECH, EMB_DIM), jnp.float32),
        pltpu.VMEM((ECH, EMB_DIM), jnp.float32),
        pltpu.SemaphoreType.DMA,
        pltpu.SemaphoreType.DMA,
    ],
    compiler_params=pltpu.CompilerParams(use_tc_tiling_on_sc=False),
)(_emb_body)


def _head_gather_body(ihe, iho, src_enc, *refs):
    outs = refs[:8]
    idx_v, he_v, ho_v, gsem, wsem = refs[8:]
    wid = lax.axis_index("s") * 2 + lax.axis_index("c")
    pbase = wid * (TPW // 2)
    staged = []
    for j in range(NCHUNK):
        prows = pl.ds(pbase + j * ECH, ECH)
        staged.append(pltpu.async_copy(ihe.at[prows], idx_v.at[2 * j], gsem))
        staged.append(pltpu.async_copy(iho.at[prows], idx_v.at[2 * j + 1],
                                       gsem))
    for h in staged:
        h.wait()
    for j in range(NCHUNK):
        prows = pl.ds(pbase + j * ECH, ECH)
        g = (pltpu.async_copy(src_enc.at[idx_v.at[2 * j]], he_v, gsem),
             pltpu.async_copy(src_enc.at[idx_v.at[2 * j + 1]], ho_v, gsem))
        for h in g:
            h.wait()
        w = []
        for k in range(4):
            w.append(pltpu.async_copy(
                he_v.at[:, pl.ds(k * PAIR, PAIR)], outs[k].at[prows], wsem))
            w.append(pltpu.async_copy(
                ho_v.at[:, pl.ds(k * PAIR, PAIR)], outs[4 + k].at[prows],
                wsem))
        for h in w:
            h.wait()


_head_gather = functools.partial(
    pl.kernel,
    mesh=plsc.VectorSubcoreMesh(core_axis_name="c", subcore_axis_name="s"),
    out_type=tuple(
        jax.ShapeDtypeStruct((HALF, PAIR), jnp.float32) for _ in range(8)),
    scratch_types=[
        pltpu.VMEM((8, ECH), jnp.int32),
        pltpu.VMEM((ECH, ENC_SIZE), jnp.float32),
        pltpu.VMEM((ECH, ENC_SIZE), jnp.float32),
        pltpu.SemaphoreType.DMA,
        pltpu.SemaphoreType.DMA,
    ],
)(_head_gather_body)


BM = 1024


BM2 = BM // 2


def _mm_body(gp_ref, gc_ref, gs_ref, h0, h1, h2, h3, h4, h5, h6, h7,
             wp_ref, wc_ref, ws_ref, wh_ref, b_ref, o_ref):
    ghcat = jnp.concatenate(
        [h0[...], h1[...], h2[...], h3[...],
         h4[...], h5[...], h6[...], h7[...]], axis=1)
    acc = jnp.dot(ghcat, wh_ref[...], preferred_element_type=jnp.float32)
    acc += b_ref[...]
    acc += jnp.dot(gp_ref[...], wp_ref[...],
                   preferred_element_type=jnp.float32)
    acc += jnp.dot(gc_ref[...], wc_ref[...],
                   preferred_element_type=jnp.float32)
    acc += jnp.dot(gs_ref[...], ws_ref[...],
                   preferred_element_type=jnp.float32)
    o_ref[...] = jnp.maximum(acc, 0.0).reshape(BM, REL_DIM)


def _matmul(gp, gc, gs, ghs, wp2, wc2, ws2, wh2, b2):
    pair_spec = pl.BlockSpec((BM2, PAIR), lambda i: (i, 0))
    w_spec = pl.BlockSpec((PAIR, 2 * REL_DIM), lambda i: (0, 0))
    return pl.pallas_call(
        _mm_body,
        grid=(HALF // BM2,),
        in_specs=[
            pair_spec, pair_spec, pair_spec,
            pair_spec, pair_spec, pair_spec, pair_spec,
            pair_spec, pair_spec, pair_spec, pair_spec,
            w_spec, w_spec, w_spec,
            pl.BlockSpec((2 * ENC_SIZE, 2 * REL_DIM), lambda i: (0, 0)),
            pl.BlockSpec((1, 2 * REL_DIM), lambda i: (0, 0)),
        ],
        out_specs=pl.BlockSpec((BM, REL_DIM), lambda i: (i, 0)),
        out_shape=jax.ShapeDtypeStruct((TOTAL, REL_DIM), jnp.float32),
    )(gp, gc, gs, *ghs, wp2, wc2, ws2, wh2, b2)


def _blockdiag(w):
    z = jnp.zeros((EMB_DIM, REL_DIM), jnp.float32)
    return jnp.concatenate(
        [jnp.concatenate([w, z], axis=1),
         jnp.concatenate([z, w], axis=1)], axis=0)


def kernel(input_data, index, src_enc_data, pos_table, cat_table, sense_table,
           W, b, lengths):
    ids = input_data.astype(jnp.int32)
    t = jnp.arange(TOTAL, dtype=jnp.int32)
    flat_idx = (t // SEQ_LEN) * SEQ_LEN + index.astype(jnp.int32)
    ipe, ice, ise = ids[0::2, 0], ids[0::2, 1], ids[0::2, 2]
    ipo, ico, iso = ids[1::2, 0], ids[1::2, 1], ids[1::2, 2]
    ihe, iho = flat_idx[0::2], flat_idx[1::2]
    wp = W[:EMB_DIM]
    wc = W[EMB_DIM:2 * EMB_DIM]
    ws = W[2 * EMB_DIM:3 * EMB_DIM]
    wh = W[3 * EMB_DIM:]
    gp, gc, gs = _emb_gather(ipe, ipo, ice, ico, ise, iso,
                             pos_table, cat_table, sense_table)
    ghs = _head_gather(ihe, iho, src_enc_data)
    wh2 = jnp.zeros((2 * ENC_SIZE, 2 * REL_DIM), jnp.float32)
    wh2 = wh2.at[:ENC_SIZE, :REL_DIM].set(wh).at[ENC_SIZE:, REL_DIM:].set(wh)
    return _matmul(gp, gc, gs, ghs, _blockdiag(wp), _blockdiag(wc),
                   _blockdiag(ws), wh2,
                   jnp.concatenate([b, b]).reshape(1, 2 * REL_DIM))


# speedup vs baseline: 1.2042x; 1.0407x over previous
import functools

import jax
import jax.numpy as jnp
from jax import lax
from jax.experimental import pallas as pl
from jax.experimental.pallas import tpu as pltpu
from jax.experimental.pallas import tpu_sc as plsc

BATCH = 16
SEQ_LEN = 1024
TOTAL = BATCH * SEQ_LEN
HALF = TOTAL // 2
EMB_DIM = 64
ENC_SIZE = 512
REL_DIM = 256
PAIR = 2 * EMB_DIM

NUM_WORKERS = 32
TPW = TOTAL // NUM_WORKERS
CHUNK = 128
ECH = CHUNK // 2
NCHUNK = TPW // CHUNK


def _emb_body(ipe, ipo, ice, ico, ise, iso,
              pos_t, cat_t, sense_t,
              out_pos, out_cat, out_sense,
              idx_v, pe_v, po_v, ce_v, co_v, se_v, so_v, gsem, wsem):
    wid = lax.axis_index("s") * 2 + lax.axis_index("c")
    pbase = wid * (TPW // 2)
    streams = ((ipe, ipo), (ice, ico), (ise, iso))
    staged = []
    for k, (ide, ido) in enumerate(streams):
        for j in range(NCHUNK):
            prows = pl.ds(pbase + j * ECH, ECH)
            staged.append(pltpu.async_copy(
                ide.at[prows], idx_v.at[(2 * k) * NCHUNK + j], gsem))
            staged.append(pltpu.async_copy(
                ido.at[prows], idx_v.at[(2 * k + 1) * NCHUNK + j], gsem))
    for h in staged:
        h.wait()
    bufs = ((pe_v, po_v), (ce_v, co_v), (se_v, so_v))
    tables = (pos_t, cat_t, sense_t)
    outs = (out_pos, out_cat, out_sense)
    for j in range(NCHUNK):
        prows = pl.ds(pbase + j * ECH, ECH)
        gathers = []
        for k in range(3):
            gathers.append(pltpu.async_copy(
                tables[k].at[idx_v.at[(2 * k) * NCHUNK + j]], bufs[k][0], gsem))
            gathers.append(pltpu.async_copy(
                tables[k].at[idx_v.at[(2 * k + 1) * NCHUNK + j]], bufs[k][1],
                gsem))
        for h in gathers:
            h.wait()
        writes = []
        for k in range(3):
            writes.append(pltpu.async_copy(
                bufs[k][0], outs[k].at[prows, pl.ds(0, EMB_DIM)], wsem))
            writes.append(pltpu.async_copy(
                bufs[k][1], outs[k].at[prows, pl.ds(EMB_DIM, EMB_DIM)], wsem))
        for h in writes:
            h.wait()


_emb_gather = functools.partial(
    pl.kernel,
    mesh=plsc.VectorSubcoreMesh(core_axis_name="c", subcore_axis_name="s"),
    out_type=(
        jax.ShapeDtypeStruct((HALF, PAIR), jnp.float32),
        jax.ShapeDtypeStruct((HALF, PAIR), jnp.float32),
        jax.ShapeDtypeStruct((HALF, PAIR), jnp.float32),
    ),
    scratch_types=[
        pltpu.VMEM((24, ECH), jnp.int32),
        pltpu.VMEM((ECH, EMB_DIM), jnp.float32),
        pltpu.VMEM((ECH, EMB_DIM), jnp.float32),
        pltpu.VMEM((ECH, EMB_DIM), jnp.float32),
        pltpu.VMEM((ECH, EMB_DIM), jnp.float32),
        pltpu.VMEM((ECH, EMB_DIM), jnp.float32),
        pltpu.VMEM((ECH, EMB_DIM), jnp.float32),
        pltpu.SemaphoreType.DMA,
        pltpu.SemaphoreType.DMA,
    ],
    compiler_params=pltpu.CompilerParams(use_tc_tiling_on_sc=False),
)(_emb_body)


BM = 1024


BM2 = BM // 2


def _mm_body(gp_ref, gc_ref, gs_ref, idx_ref, src_ref,
             wp_ref, wc_ref, ws_ref, wh_ref, b_ref, o_ref):
    emb = jnp.dot(gp_ref[...], wp_ref[...],
                  preferred_element_type=jnp.float32)
    emb += jnp.dot(gc_ref[...], wc_ref[...],
                   preferred_element_type=jnp.float32)
    emb += jnp.dot(gs_ref[...], ws_ref[...],
                   preferred_element_type=jnp.float32)
    acc = emb.reshape(BM, REL_DIM) + b_ref[...]
    pos = jax.lax.broadcasted_iota(jnp.int32, (BM, SEQ_LEN), 1)
    onehot = jnp.where(pos == idx_ref[...], 1.0, 0.0).astype(jnp.bfloat16)
    head = jnp.dot(onehot, src_ref[...].astype(jnp.bfloat16),
                   preferred_element_type=jnp.float32)
    acc += jnp.dot(head, wh_ref[...], preferred_element_type=jnp.float32)
    o_ref[...] = jnp.maximum(acc, 0.0)


def _matmul(gp, gc, gs, idx2d, src_enc, wp2, wc2, ws2, wh, b2):
    pair_spec = pl.BlockSpec((BM2, PAIR), lambda i: (i, 0))
    w_spec = pl.BlockSpec((PAIR, 2 * REL_DIM), lambda i: (0, 0))
    return pl.pallas_call(
        _mm_body,
        grid=(HALF // BM2,),
        in_specs=[
            pair_spec, pair_spec, pair_spec,
            pl.BlockSpec((BM, 1), lambda i: (i, 0)),
            pl.BlockSpec((BM, ENC_SIZE), lambda i: (i, 0)),
            w_spec, w_spec, w_spec,
            pl.BlockSpec((ENC_SIZE, REL_DIM), lambda i: (0, 0)),
            pl.BlockSpec((1, REL_DIM), lambda i: (0, 0)),
        ],
        out_specs=pl.BlockSpec((BM, REL_DIM), lambda i: (i, 0)),
        out_shape=jax.ShapeDtypeStruct((TOTAL, REL_DIM), jnp.float32),
    )(gp, gc, gs, idx2d, src_enc, wp2, wc2, ws2, wh, b2)


def _blockdiag(w):
    z = jnp.zeros((EMB_DIM, REL_DIM), jnp.float32)
    return jnp.concatenate(
        [jnp.concatenate([w, z], axis=1),
         jnp.concatenate([z, w], axis=1)], axis=0)


def kernel(input_data, index, src_enc_data, pos_table, cat_table, sense_table,
           W, b, lengths):
    ids = input_data.astype(jnp.int32)
    t = jnp.arange(TOTAL, dtype=jnp.int32)
    flat_idx = (t // SEQ_LEN) * SEQ_LEN + index.astype(jnp.int32)
    ipe, ice, ise = ids[0::2, 0], ids[0::2, 1], ids[0::2, 2]
    ipo, ico, iso = ids[1::2, 0], ids[1::2, 1], ids[1::2, 2]
    ihe, iho = flat_idx[0::2], flat_idx[1::2]
    wp = W[:EMB_DIM]
    wc = W[EMB_DIM:2 * EMB_DIM]
    ws = W[2 * EMB_DIM:3 * EMB_DIM]
    wh = W[3 * EMB_DIM:]
    gp, gc, gs = _emb_gather(ipe, ipo, ice, ico, ise, iso,
                             pos_table, cat_table, sense_table)
    idx2d = index.astype(jnp.int32).reshape(TOTAL, 1)
    return _matmul(gp, gc, gs, idx2d, src_enc_data, _blockdiag(wp),
                   _blockdiag(wc), _blockdiag(ws), wh,
                   b.reshape(1, REL_DIM))
